# Initial kernel scaffold; baseline (speedup 1.0000x reference)
import jax
import jax.numpy as jnp
from jax.experimental import pallas as pl


def kernel(encoder_hidden, decoder_hidden, relative_attention_bias):
    raise NotImplementedError("write your pallas kernel here")


# speedup vs baseline: 42.3213x; 42.3213x over previous
import functools
import math

import jax
import jax.numpy as jnp
from jax import lax
from jax.experimental import pallas as pl
from jax.experimental.pallas import tpu as pltpu
from jax.experimental.pallas import tpu_sc as plsc

_NUM_BUCKETS = 32
_MAX_DISTANCE = 128
_HEADS = 16
_Q = 2048
_K = 2048
_NSHIFT = 8
_TAB = 4096


def _shift_table_kernel(bias_ref, out_ref):
    r = lax.broadcasted_iota(jnp.int32, (_NSHIFT, _TAB), 0)
    j = lax.broadcasted_iota(jnp.int32, (_NSHIFT, _TAB), 1)
    relative_position = j + r - (_Q - 1)

    num_buckets = _NUM_BUCKETS // 2
    relative_buckets = (relative_position > 0).astype(jnp.int32) * num_buckets
    n = jnp.abs(relative_position)
    max_exact = num_buckets // 2
    is_small = n < max_exact
    nf = n.astype(jnp.float32)
    rp_if_large = max_exact + jnp.log(nf / max_exact) / math.log(
        _MAX_DISTANCE / max_exact
    ) * (num_buckets - max_exact)
    rp_if_large = jnp.minimum(
        rp_if_large, jnp.full_like(rp_if_large, num_buckets - 1)
    )
    buckets_f = relative_buckets.astype(jnp.float32) + jnp.where(
        is_small, nf, rp_if_large
    )
    bucket = buckets_f.astype(jnp.int32)

    vals = jnp.zeros((_NSHIFT, _TAB), jnp.float32)
    for b in range(_NUM_BUCKETS):
        vals = jnp.where(bucket == b, bias_ref[0, 0, b], vals)
    out_ref[0] = vals


def _build_shift_tables(relative_attention_bias):
    bias_t = relative_attention_bias.T.reshape(_HEADS, 1, _NUM_BUCKETS)
    shift = pl.pallas_call(
        _shift_table_kernel,
        grid=(_HEADS,),
        in_specs=[pl.BlockSpec((1, 1, _NUM_BUCKETS), lambda h: (h, 0, 0))],
        out_specs=pl.BlockSpec((1, _NSHIFT, _TAB), lambda h: (h, 0, 0)),
        out_shape=jax.ShapeDtypeStruct((_HEADS, _NSHIFT, _TAB), jnp.float32),
    )(bias_t)
    return shift.reshape(_HEADS * _NSHIFT * _TAB)


_ROWS_PER_WORKER = _Q // 2
_CHUNK = 8


def _expand_kernel(shift_hbm, out_hbm, tab_vmem, sem):
    c = lax.axis_index("c")
    s = lax.axis_index("s")
    h = s
    qbase = c * _ROWS_PER_WORKER
    rowbase = h * _Q + qbase

    hoff = pl.multiple_of(h * (_NSHIFT * _TAB), 8)
    pltpu.sync_copy(shift_hbm.at[pl.ds(hoff, _NSHIFT * _TAB)], tab_vmem)

    def issue(qi):
        q = qbase + qi
        r = 7 - lax.rem(q, 8)
        m = 255 - lax.div(q, 8)
        woff = pl.multiple_of(r * _TAB + 8 * m, 8)
        doff = pl.multiple_of((rowbase + qi) * _K, 8)
        pltpu.make_async_copy(
            tab_vmem.at[pl.ds(woff, _K)],
            out_hbm.at[pl.ds(doff, _K)],
            sem,
        ).start()

    def drain_one():
        pltpu.make_async_copy(
            tab_vmem.at[pl.ds(0, _K)],
            out_hbm.at[pl.ds(rowbase * _K, _K)],
            sem,
        ).wait()

    for jj in range(_CHUNK):
        issue(jj)

    def body(i, carry):
        base = i * _CHUNK
        for jj in range(_CHUNK):
            issue(base + jj)
        for jj in range(_CHUNK):
            drain_one()
        return carry

    lax.fori_loop(1, _ROWS_PER_WORKER // _CHUNK, body, 0)
    for jj in range(_CHUNK):
        drain_one()


def kernel(encoder_hidden, decoder_hidden, relative_attention_bias):
    del encoder_hidden, decoder_hidden
    shift = _build_shift_tables(relative_attention_bias)

    mesh = plsc.VectorSubcoreMesh(core_axis_name="c", subcore_axis_name="s")
    expand = functools.partial(
        pl.kernel,
        mesh=mesh,
        out_type=jax.ShapeDtypeStruct((_HEADS * _Q * _K,), jnp.float32),
        scratch_types=[
            pltpu.VMEM((_NSHIFT * _TAB,), jnp.float32),
            pltpu.SemaphoreType.DMA,
        ],
    )(_expand_kernel)
    out = expand(shift)
    return out.reshape(1, _HEADS, _Q, _K)
